# Initial kernel scaffold; baseline (speedup 1.0000x reference)
import functools

import jax
import jax.numpy as jnp
from jax import lax
from jax.experimental import pallas as pl
from jax.experimental.pallas import tpu as pltpu
from jax.experimental.pallas import tpu_sc as plsc

NC = 2
NS = 16
NW = NC * NS
F = 64
CW = 80
PW = 32


def _dgt(a, b):
    return lax.dot_general(a, b, (((1,), (1,)), ((), ())),
                           preferred_element_type=jnp.float32)


def _mesh():
    return plsc.VectorSubcoreMesh(core_axis_name="c", subcore_axis_name="s")


def _proj_body(x_ref, wl_ref, bl_ref, wr_ref, br_ref, xl_ref, xr_ref):
    xb = x_ref[...]
    xl_ref[...] = _dgt(xb, wl_ref[...]) + bl_ref[...]
    xr_ref[...] = _dgt(xb, wr_ref[...]) + br_ref[...]


def _proj(x, Wl, bl, Wr, br, blk=1000):
    n, k = x.shape
    f = Wl.shape[0]
    return pl.pallas_call(
        _proj_body,
        grid=(n // blk,),
        in_specs=[
            pl.BlockSpec((blk, k), lambda i: (i, 0)),
            pl.BlockSpec((f, k), lambda i: (0, 0)),
            pl.BlockSpec((1, f), lambda i: (0, 0)),
            pl.BlockSpec((f, k), lambda i: (0, 0)),
            pl.BlockSpec((1, f), lambda i: (0, 0)),
        ],
        out_specs=[pl.BlockSpec((blk, f), lambda i: (i, 0)),
                   pl.BlockSpec((blk, f), lambda i: (i, 0))],
        out_shape=[jax.ShapeDtypeStruct((n, f), jnp.float32)] * 2,
    )(x, Wl, bl.reshape(1, -1), Wr, br.reshape(1, -1))


def _me_body(ea_ref, w1_ref, w2_ref, m1_ref, m2_ref):
    ea = ea_ref[...]
    m1_ref[...] = _dgt(ea, w1_ref[...])
    m2_ref[...] = _dgt(ea, w2_ref[...])


def _me2(ea, We1, We2, blk=4000):
    e, k = ea.shape
    f = We1.shape[0]
    return pl.pallas_call(
        _me_body,
        grid=(e // blk,),
        in_specs=[
            pl.BlockSpec((blk, k), lambda i: (i, 0)),
            pl.BlockSpec((f, k), lambda i: (0, 0)),
            pl.BlockSpec((f, k), lambda i: (0, 0)),
        ],
        out_specs=[pl.BlockSpec((blk, f), lambda i: (i, 0)),
                   pl.BlockSpec((blk, f), lambda i: (i, 0))],
        out_shape=[jax.ShapeDtypeStruct((e, f), jnp.float32)] * 2,
    )(ea, We1, We2)


def _loopme_body(pacc_ref, w1_ref, w2_ref, l1_ref, l2_ref):
    p = pacc_ref[...]
    s = p[0] + p[1]
    cnt = jnp.clip(s[:, 16:17], 1.0, None)
    la = s[:, :16] / cnt
    l1_ref[...] = _dgt(la, w1_ref[...])
    l2_ref[...] = _dgt(la, w2_ref[...])


def _loopme(pacc, We1, We2, blk=1000):
    n = pacc.shape[1]
    f = We1.shape[0]
    return pl.pallas_call(
        _loopme_body,
        grid=(n // blk,),
        in_specs=[
            pl.BlockSpec((2, blk, PW), lambda i: (0, i, 0)),
            pl.BlockSpec((f, 16), lambda i: (0, 0)),
            pl.BlockSpec((f, 16), lambda i: (0, 0)),
        ],
        out_specs=[pl.BlockSpec((blk, f), lambda i: (i, 0)),
                   pl.BlockSpec((blk, f), lambda i: (i, 0))],
        out_shape=[jax.ShapeDtypeStruct((n, f), jnp.float32)] * 2,
    )(pacc, We1, We2)


def _node_h(acc, xl, xr, lme, att, bias):
    ms = xl + xr + lme
    ms = jnp.where(ms > 0, ms, 0.2 * ms)
    aes = jnp.exp(jnp.sum(ms * att, axis=-1, keepdims=True))
    num = acc[0, :, :F] + acc[1, :, :F] + xl * aes
    den = acc[0, :, F:F + 1] + acc[1, :, F:F + 1] + aes + 1e-16
    return jnp.maximum(num / den + bias, 0.0)


def _epi_body(acc_ref, xl_ref, xr_ref, lme_ref, att_ref, bias_ref,
              wl_ref, bl_ref, wr_ref, br_ref, xl2_ref, xr2_ref):
    h = _node_h(acc_ref[...], xl_ref[...], xr_ref[...], lme_ref[...],
                att_ref[...], bias_ref[...])
    xl2_ref[...] = _dgt(h, wl_ref[...]) + bl_ref[...]
    xr2_ref[...] = _dgt(h, wr_ref[...]) + br_ref[...]


def _epi(acc, xl, xr, lme, att, bias, Wl, bl, Wr, br, blk=1000):
    n = xl.shape[0]
    f = F
    return pl.pallas_call(
        _epi_body,
        grid=(n // blk,),
        in_specs=[
            pl.BlockSpec((2, blk, CW), lambda i: (0, i, 0)),
            pl.BlockSpec((blk, f), lambda i: (i, 0)),
            pl.BlockSpec((blk, f), lambda i: (i, 0)),
            pl.BlockSpec((blk, f), lambda i: (i, 0)),
            pl.BlockSpec((1, f), lambda i: (0, 0)),
            pl.BlockSpec((1, f), lambda i: (0, 0)),
            pl.BlockSpec((f, f), lambda i: (0, 0)),
            pl.BlockSpec((1, f), lambda i: (0, 0)),
            pl.BlockSpec((f, f), lambda i: (0, 0)),
            pl.BlockSpec((1, f), lambda i: (0, 0)),
        ],
        out_specs=[pl.BlockSpec((blk, f), lambda i: (i, 0)),
                   pl.BlockSpec((blk, f), lambda i: (i, 0))],
        out_shape=[jax.ShapeDtypeStruct((n, f), jnp.float32)] * 2,
    )(acc, xl, xr, lme, att.reshape(1, -1), bias.reshape(1, -1),
      Wl, bl.reshape(1, -1), Wr, br.reshape(1, -1))


def _head_body(acc_ref, xl_ref, xr_ref, lme_ref, att_ref, bias_ref, y_ref,
               w0_ref, b0_ref, w1a_ref, w1b_ref, b1_ref, w2_ref, b2_ref,
               out_ref):
    h2 = _node_h(acc_ref[...], xl_ref[...], xr_ref[...], lme_ref[...],
                 att_ref[...], bias_ref[...])
    y2 = jnp.maximum(_dgt(y_ref[...], w0_ref[...]) + b0_ref[...], 0.0)
    hc = jnp.maximum(_dgt(h2, w1a_ref[...]) + _dgt(y2, w1b_ref[...])
                     + b1_ref[...], 0.0)
    o = _dgt(hc, w2_ref[...]) + b2_ref[...]
    out_ref[...] = jnp.concatenate([o, jnp.zeros_like(o)], axis=1)


def _head(acc, xl, xr, lme, att, bias, y, W0, b0, W1, b1, W2, b2, blk=1000):
    n = xl.shape[0]
    f = F
    W1a = W1[:, :f]
    W1b = W1[:, f:]
    return pl.pallas_call(
        _head_body,
        grid=(n // blk,),
        in_specs=[
            pl.BlockSpec((2, blk, CW), lambda i: (0, i, 0)),
            pl.BlockSpec((blk, f), lambda i: (i, 0)),
            pl.BlockSpec((blk, f), lambda i: (i, 0)),
            pl.BlockSpec((blk, f), lambda i: (i, 0)),
            pl.BlockSpec((1, f), lambda i: (0, 0)),
            pl.BlockSpec((1, f), lambda i: (0, 0)),
            pl.BlockSpec((blk, 2), lambda i: (i, 0)),
            pl.BlockSpec((2, 2), lambda i: (0, 0)),
            pl.BlockSpec((1, 2), lambda i: (0, 0)),
            pl.BlockSpec((32, f), lambda i: (0, 0)),
            pl.BlockSpec((32, 2), lambda i: (0, 0)),
            pl.BlockSpec((1, 32), lambda i: (0, 0)),
            pl.BlockSpec((8, 32), lambda i: (0, 0)),
            pl.BlockSpec((1, 8), lambda i: (0, 0)),
        ],
        out_specs=[pl.BlockSpec((blk, 16), lambda i: (i, 0))],
        out_shape=[jax.ShapeDtypeStruct((n, 16), jnp.float32)],
    )(acc, xl, xr, lme, att.reshape(1, -1), bias.reshape(1, -1),
      y, W0, b0.reshape(1, -1), W1a, W1b, b1.reshape(1, -1),
      W2, b2.reshape(1, -1))[0]


def _p0(dst, ea, zeros, n):
    e = dst.shape[0]
    ept = e // NW
    b0 = 400
    npt = n // NS

    @functools.partial(
        pl.kernel,
        out_type=jax.ShapeDtypeStruct((2, n, PW), jnp.float32),
        mesh=_mesh(),
        scratch_types=[
            pltpu.VMEM((b0,), jnp.int32),
            pltpu.VMEM((b0, 16), jnp.float32),
            pltpu.VMEM((b0, PW), jnp.float32),
            pltpu.VMEM_SHARED((n, PW), jnp.float32),
        ],
    )
    def k(dst_h, ea_h, z_h, out_h, idx_v, ea_v, con_v, acc_s):
        cid = lax.axis_index("c")
        sid = lax.axis_index("s")
        wid = cid * NS + sid
        r0 = sid * npt
        pltpu.sync_copy(z_h.at[pl.ds(r0, npt)], acc_s.at[pl.ds(r0, npt)])
        ones = jnp.ones((16,), jnp.float32)
        iota = lax.iota(jnp.int32, 16)
        col16 = jnp.full((16,), 16, jnp.int32)

        def initg(g, _):
            plsc.store_scatter(con_v, [g * 16 + iota, col16], ones)
            return 0

        lax.fori_loop(0, b0 // 16, initg, 0)
        plsc.subcore_barrier()

        def blk(b, _):
            eb = pl.multiple_of(wid * ept + b * b0, 8)
            pltpu.sync_copy(dst_h.at[pl.ds(eb, b0)], idx_v)
            pltpu.sync_copy(ea_h.at[pl.ds(eb, b0)], ea_v)

            def cp(r, _):
                con_v[r, pl.ds(0, 16)] = ea_v[r, :]
                return 0

            lax.fori_loop(0, b0, cp, 0)
            pltpu.sync_copy(con_v, acc_s.at[idx_v], add=True)
            return 0

        lax.fori_loop(0, ept // b0, blk, 0)
        plsc.subcore_barrier()
        pltpu.sync_copy(acc_s.at[pl.ds(r0, npt)], out_h.at[cid, pl.ds(r0, npt)])

    return k(dst, ea, zeros)


def _edge_pass(xl, xr, me, src, dst, att, zeros):
    n = xl.shape[0]
    e = src.shape[0]
    ept = e // NW
    bsz = 80
    grp = bsz // 16
    npt = n // NS

    @functools.partial(
        pl.kernel,
        out_type=jax.ShapeDtypeStruct((2, n, CW), jnp.float32),
        mesh=_mesh(),
        scratch_types=[
            pltpu.VMEM((bsz,), jnp.int32),
            pltpu.VMEM((bsz,), jnp.int32),
            pltpu.VMEM((bsz, F), jnp.float32),
            pltpu.VMEM((bsz, F), jnp.float32),
            pltpu.VMEM((bsz, F), jnp.float32),
            pltpu.VMEM((bsz, CW), jnp.float32),
            pltpu.VMEM((F,), jnp.float32),
            pltpu.VMEM((16,), jnp.float32),
            pltpu.VMEM_SHARED((n, CW), jnp.float32),
            pltpu.SemaphoreType.DMA,
            pltpu.SemaphoreType.DMA,
        ],
    )
    def k(xl_h, xr_h, me_h, src_h, dst_h, att_h, z_h, out_h,
          sidx_v, didx_v, xl_v, xr_v, me_v, con_v, att_v, ae_v,
          acc_s, sem1, sem2):
        cid = lax.axis_index("c")
        sid = lax.axis_index("s")
        wid = cid * NS + sid
        r0 = sid * npt
        pltpu.sync_copy(z_h.at[pl.ds(r0, npt)], acc_s.at[pl.ds(r0, npt)])
        pltpu.sync_copy(att_h, att_v)
        plsc.subcore_barrier()
        iota = lax.iota(jnp.int32, 16)
        zero16 = jnp.zeros((16,), jnp.float32)
        col64 = jnp.full((16,), F, jnp.int32)

        def blk(b, _):
            eb = pl.multiple_of(wid * ept + b * bsz, 8)
            pltpu.sync_copy(src_h.at[pl.ds(eb, bsz)], sidx_v)
            pltpu.sync_copy(dst_h.at[pl.ds(eb, bsz)], didx_v)
            g1 = pltpu.async_copy(xl_h.at[sidx_v], xl_v, sem1)
            g2 = pltpu.async_copy(xr_h.at[didx_v], xr_v, sem2)
            pltpu.sync_copy(me_h.at[pl.ds(eb, bsz)], me_v)
            g1.wait()
            g2.wait()

            def jloop(j, accs):
                colj = jnp.full((16,), j, jnp.int32)
                attj = plsc.load_gather(att_v, [colj])
                out = []
                for g in range(grp):
                    rows = g * 16 + iota
                    mm = (plsc.load_gather(xl_v, [rows, colj])
                          + plsc.load_gather(xr_v, [rows, colj])
                          + plsc.load_gather(me_v, [rows, colj]))
                    mm = jnp.where(mm > 0, mm, 0.2 * mm)
                    out.append(accs[g] + mm * attj)
                return tuple(out)

            accs = lax.fori_loop(0, F, jloop, (zero16,) * grp)
            for g in range(grp):
                ae = jnp.exp(accs[g])
                ae_v[...] = ae

                def eloop(ei, _):
                    r = g * 16 + ei
                    bc = plsc.load_gather(ae_v, [jnp.full((16,), ei, jnp.int32)])
                    for kk in range(F // 16):
                        con_v[r, pl.ds(kk * 16, 16)] = (
                            xl_v[r, pl.ds(kk * 16, 16)] * bc)
                    return 0

                lax.fori_loop(0, 16, eloop, 0)
                plsc.store_scatter(con_v, [g * 16 + iota, col64], ae)
            pltpu.sync_copy(con_v, acc_s.at[didx_v], add=True)
            return 0

        lax.fori_loop(0, ept // bsz, blk, 0)
        plsc.subcore_barrier()
        pltpu.sync_copy(acc_s.at[pl.ds(r0, npt)], out_h.at[cid, pl.ds(r0, npt)])

    return k(xl, xr, me, src, dst, att, zeros)


def _gather_rows(tab, idx):
    kn = idx.shape[0]
    kpt = kn // NW

    @functools.partial(
        pl.kernel,
        out_type=jax.ShapeDtypeStruct((kn, 16), jnp.float32),
        mesh=_mesh(),
        scratch_types=[
            pltpu.VMEM((kpt,), jnp.int32),
            pltpu.VMEM((kpt, 16), jnp.float32),
            pltpu.SemaphoreType.DMA,
        ],
    )
    def k(tab_h, idx_h, out_h, idx_v, rows_v, sem):
        cid = lax.axis_index("c")
        sid = lax.axis_index("s")
        base = (cid * NS + sid) * kpt
        pltpu.sync_copy(idx_h.at[pl.ds(base, kpt)], idx_v)
        pltpu.async_copy(tab_h.at[idx_v], rows_v, sem).wait()
        pltpu.sync_copy(rows_v, out_h.at[pl.ds(base, kpt)])

    return k(tab, idx)


def kernel(x, edge_index, edge_attr, y, node_idx,
           Wl1, bl1, Wr1, br1, We1, att1, bias1,
           Wl2, bl2, Wr2, br2, We2, att2, bias2,
           W0, b0, W1, b1, W2, b2):
    n = x.shape[0]
    src = edge_index[0].astype(jnp.int32)
    dst = edge_index[1].astype(jnp.int32)
    node_idx = node_idx.astype(jnp.int32)

    zeros_cw = jnp.zeros((n, CW), jnp.float32)
    zeros_pw = jnp.zeros((n, PW), jnp.float32)

    xl1, xr1 = _proj(x, Wl1, bl1, Wr1, br1)
    me1, me2 = _me2(edge_attr, We1, We2)
    pacc = _p0(dst, edge_attr, zeros_pw, n)
    lme1, lme2 = _loopme(pacc, We1, We2)

    acc1 = _edge_pass(xl1, xr1, me1, src, dst, att1, zeros_cw)
    xl2, xr2 = _epi(acc1, xl1, xr1, lme1, att1, bias1, Wl2, bl2, Wr2, br2)
    acc2 = _edge_pass(xl2, xr2, me2, src, dst, att2, zeros_cw)
    outp = _head(acc2, xl2, xr2, lme2, att2, bias2, y, W0, b0, W1, b1, W2, b2)
    sel = _gather_rows(outp, node_idx)
    return sel[:, :8]


# speedup vs baseline: 4.1561x; 4.1561x over previous
import functools

import jax
import jax.numpy as jnp
from jax import lax
from jax.experimental import pallas as pl
from jax.experimental.pallas import tpu as pltpu
from jax.experimental.pallas import tpu_sc as plsc

NC = 2
NS = 16
NW = NC * NS
F = 64
CW = 80
PW = 32


def _dgt(a, b):
    return lax.dot_general(a, b, (((1,), (1,)), ((), ())),
                           preferred_element_type=jnp.float32)


def _mesh():
    return plsc.VectorSubcoreMesh(core_axis_name="c", subcore_axis_name="s")


def _proj_body(x_ref, wl_ref, bl_ref, wr_ref, br_ref, xl_ref, xr_ref):
    xb = x_ref[...]
    xl_ref[...] = _dgt(xb, wl_ref[...]) + bl_ref[...]
    xr_ref[...] = _dgt(xb, wr_ref[...]) + br_ref[...]


def _proj(x, Wl, bl, Wr, br, blk=1000):
    n, k = x.shape
    f = Wl.shape[0]
    return pl.pallas_call(
        _proj_body,
        grid=(n // blk,),
        in_specs=[
            pl.BlockSpec((blk, k), lambda i: (i, 0)),
            pl.BlockSpec((f, k), lambda i: (0, 0)),
            pl.BlockSpec((1, f), lambda i: (0, 0)),
            pl.BlockSpec((f, k), lambda i: (0, 0)),
            pl.BlockSpec((1, f), lambda i: (0, 0)),
        ],
        out_specs=[pl.BlockSpec((blk, f), lambda i: (i, 0)),
                   pl.BlockSpec((blk, f), lambda i: (i, 0))],
        out_shape=[jax.ShapeDtypeStruct((n, f), jnp.float32)] * 2,
    )(x, Wl, bl.reshape(1, -1), Wr, br.reshape(1, -1))


def _me_body(ea_ref, w1_ref, w2_ref, m1_ref, m2_ref):
    ea = ea_ref[...]
    m1_ref[...] = _dgt(ea, w1_ref[...])
    m2_ref[...] = _dgt(ea, w2_ref[...])


def _me2(ea, We1, We2, blk=4000):
    e, k = ea.shape
    f = We1.shape[0]
    return pl.pallas_call(
        _me_body,
        grid=(e // blk,),
        in_specs=[
            pl.BlockSpec((blk, k), lambda i: (i, 0)),
            pl.BlockSpec((f, k), lambda i: (0, 0)),
            pl.BlockSpec((f, k), lambda i: (0, 0)),
        ],
        out_specs=[pl.BlockSpec((blk, f), lambda i: (i, 0)),
                   pl.BlockSpec((blk, f), lambda i: (i, 0))],
        out_shape=[jax.ShapeDtypeStruct((e, f), jnp.float32)] * 2,
    )(ea, We1, We2)


def _loopme_body(pacc_ref, w1_ref, w2_ref, l1_ref, l2_ref):
    p = pacc_ref[...]
    s = p[0] + p[1]
    cnt = jnp.clip(s[:, 16:17], 1.0, None)
    la = s[:, :16] / cnt
    l1_ref[...] = _dgt(la, w1_ref[...])
    l2_ref[...] = _dgt(la, w2_ref[...])


def _loopme(pacc, We1, We2, n, blk=1000):
    f = We1.shape[0]
    return pl.pallas_call(
        _loopme_body,
        grid=(n // blk,),
        in_specs=[
            pl.BlockSpec((2, blk, PW), lambda i: (0, i, 0)),
            pl.BlockSpec((f, 16), lambda i: (0, 0)),
            pl.BlockSpec((f, 16), lambda i: (0, 0)),
        ],
        out_specs=[pl.BlockSpec((blk, f), lambda i: (i, 0)),
                   pl.BlockSpec((blk, f), lambda i: (i, 0))],
        out_shape=[jax.ShapeDtypeStruct((n, f), jnp.float32)] * 2,
    )(pacc, We1, We2)


def _node_h(acc, xl, xr, lme, att, bias):
    ms = xl + xr + lme
    ms = jnp.where(ms > 0, ms, 0.2 * ms)
    aes = jnp.exp(jnp.sum(ms * att, axis=-1, keepdims=True))
    num = acc[0, :, :F] + acc[1, :, :F] + xl * aes
    den = acc[0, :, F:F + 1] + acc[1, :, F:F + 1] + aes + 1e-16
    return jnp.maximum(num / den + bias, 0.0)


def _epi_body(acc_ref, xl_ref, xr_ref, lme_ref, att_ref, bias_ref,
              wl_ref, bl_ref, wr_ref, br_ref, xl2_ref, xr2_ref):
    h = _node_h(acc_ref[...], xl_ref[...], xr_ref[...], lme_ref[...],
                att_ref[...], bias_ref[...])
    xl2_ref[...] = _dgt(h, wl_ref[...]) + bl_ref[...]
    xr2_ref[...] = _dgt(h, wr_ref[...]) + br_ref[...]


def _epi(acc, xl, xr, lme, att, bias, Wl, bl, Wr, br, blk=1000):
    n = xl.shape[0]
    f = F
    return pl.pallas_call(
        _epi_body,
        grid=(n // blk,),
        in_specs=[
            pl.BlockSpec((2, blk, CW), lambda i: (0, i, 0)),
            pl.BlockSpec((blk, f), lambda i: (i, 0)),
            pl.BlockSpec((blk, f), lambda i: (i, 0)),
            pl.BlockSpec((blk, f), lambda i: (i, 0)),
            pl.BlockSpec((1, f), lambda i: (0, 0)),
            pl.BlockSpec((1, f), lambda i: (0, 0)),
            pl.BlockSpec((f, f), lambda i: (0, 0)),
            pl.BlockSpec((1, f), lambda i: (0, 0)),
            pl.BlockSpec((f, f), lambda i: (0, 0)),
            pl.BlockSpec((1, f), lambda i: (0, 0)),
        ],
        out_specs=[pl.BlockSpec((blk, f), lambda i: (i, 0)),
                   pl.BlockSpec((blk, f), lambda i: (i, 0))],
        out_shape=[jax.ShapeDtypeStruct((n, f), jnp.float32)] * 2,
    )(acc, xl, xr, lme, att.reshape(1, -1), bias.reshape(1, -1),
      Wl, bl.reshape(1, -1), Wr, br.reshape(1, -1))


def _head_body(acc_ref, xl_ref, xr_ref, lme_ref, att_ref, bias_ref, y_ref,
               w0_ref, b0_ref, w1a_ref, w1b_ref, b1_ref, w2_ref, b2_ref,
               out_ref):
    h2 = _node_h(acc_ref[...], xl_ref[...], xr_ref[...], lme_ref[...],
                 att_ref[...], bias_ref[...])
    y2 = jnp.maximum(_dgt(y_ref[...], w0_ref[...]) + b0_ref[...], 0.0)
    hc = jnp.maximum(_dgt(h2, w1a_ref[...]) + _dgt(y2, w1b_ref[...])
                     + b1_ref[...], 0.0)
    o = _dgt(hc, w2_ref[...]) + b2_ref[...]
    out_ref[...] = jnp.concatenate([o, jnp.zeros_like(o)], axis=1)


def _head(acc, xl, xr, lme, att, bias, y, W0, b0, W1, b1, W2, b2, blk=1000):
    n = xl.shape[0]
    f = F
    W1a = W1[:, :f]
    W1b = W1[:, f:]
    return pl.pallas_call(
        _head_body,
        grid=(n // blk,),
        in_specs=[
            pl.BlockSpec((2, blk, CW), lambda i: (0, i, 0)),
            pl.BlockSpec((blk, f), lambda i: (i, 0)),
            pl.BlockSpec((blk, f), lambda i: (i, 0)),
            pl.BlockSpec((blk, f), lambda i: (i, 0)),
            pl.BlockSpec((1, f), lambda i: (0, 0)),
            pl.BlockSpec((1, f), lambda i: (0, 0)),
            pl.BlockSpec((blk, 2), lambda i: (i, 0)),
            pl.BlockSpec((2, 2), lambda i: (0, 0)),
            pl.BlockSpec((1, 2), lambda i: (0, 0)),
            pl.BlockSpec((32, f), lambda i: (0, 0)),
            pl.BlockSpec((32, 2), lambda i: (0, 0)),
            pl.BlockSpec((1, 32), lambda i: (0, 0)),
            pl.BlockSpec((8, 32), lambda i: (0, 0)),
            pl.BlockSpec((1, 8), lambda i: (0, 0)),
        ],
        out_specs=[pl.BlockSpec((blk, 16), lambda i: (i, 0))],
        out_shape=[jax.ShapeDtypeStruct((n, 16), jnp.float32)],
    )(acc, xl, xr, lme, att.reshape(1, -1), bias.reshape(1, -1),
      y, W0, b0.reshape(1, -1), W1a, W1b, b1.reshape(1, -1),
      W2, b2.reshape(1, -1))[0]


def _p0(dst, ea, zeros, npad):
    e = dst.shape[0]
    ept = e // NW
    b0 = 400
    npt = npad // NS

    @functools.partial(
        pl.kernel,
        out_type=jax.ShapeDtypeStruct((2, npad, PW), jnp.float32),
        mesh=_mesh(),
        compiler_params=pltpu.CompilerParams(needs_layout_passes=False, use_tc_tiling_on_sc=False),
        scratch_types=[
            pltpu.VMEM((b0,), jnp.int32),
            pltpu.VMEM((b0, 16), jnp.float32),
            pltpu.VMEM((b0, PW), jnp.float32),
            pltpu.VMEM_SHARED((npad, PW), jnp.float32),
        ],
    )
    def k(dst_h, ea_h, z_h, out_h, idx_v, ea_v, con_v, acc_s):
        cid = lax.axis_index("c")
        sid = lax.axis_index("s")
        wid = cid * NS + sid
        r0 = pl.multiple_of(sid * npt, 8)
        pltpu.sync_copy(z_h.at[pl.ds(r0, npt)], acc_s.at[pl.ds(r0, npt)])
        iota = lax.iota(jnp.int32, 16)
        one0 = jnp.where(iota == 0, 1.0, 0.0).astype(jnp.float32)
        plsc.subcore_barrier()

        def blk(b, _):
            eb = pl.multiple_of(wid * ept + b * b0, 8)
            pltpu.sync_copy(dst_h.at[pl.ds(eb, b0)], idx_v)
            pltpu.sync_copy(ea_h.at[pl.ds(eb, b0)], ea_v)

            def cp(r, _):
                con_v[r, pl.ds(0, 16)] = ea_v[r, :]
                con_v[r, pl.ds(16, 16)] = one0
                return 0

            lax.fori_loop(0, b0, cp, 0)
            pltpu.sync_copy(con_v, acc_s.at[idx_v], add=True)
            return 0

        lax.fori_loop(0, ept // b0, blk, 0)
        plsc.subcore_barrier()
        pltpu.sync_copy(acc_s.at[pl.ds(r0, npt)], out_h.at[cid, pl.ds(r0, npt)])

    return k(dst, ea, zeros)


def _edge_pass(xl, xr, me, src, dst, att, zeros):
    npad = zeros.shape[0]
    e = src.shape[0]
    ept = e // NW
    bsz = 80
    grp = bsz // 16
    npt = npad // NS

    @functools.partial(
        pl.kernel,
        out_type=jax.ShapeDtypeStruct((2, npad, CW), jnp.float32),
        mesh=_mesh(),
        compiler_params=pltpu.CompilerParams(needs_layout_passes=False, use_tc_tiling_on_sc=False),
        scratch_types=[
            pltpu.VMEM((bsz,), jnp.int32),
            pltpu.VMEM((bsz,), jnp.int32),
            pltpu.VMEM((bsz, F), jnp.float32),
            pltpu.VMEM((bsz, F), jnp.float32),
            pltpu.VMEM((bsz, F), jnp.float32),
            pltpu.VMEM((bsz, CW), jnp.float32),
            pltpu.VMEM((F,), jnp.float32),
            pltpu.VMEM((16,), jnp.float32),
            pltpu.VMEM_SHARED((npad, CW), jnp.float32),
            pltpu.SemaphoreType.DMA,
            pltpu.SemaphoreType.DMA,
        ],
    )
    def k(xl_h, xr_h, me_h, src_h, dst_h, att_h, z_h, out_h,
          sidx_v, didx_v, xl_v, xr_v, me_v, con_v, att_v, ae_v,
          acc_s, sem1, sem2):
        cid = lax.axis_index("c")
        sid = lax.axis_index("s")
        wid = cid * NS + sid
        r0 = pl.multiple_of(sid * npt, 8)
        pltpu.sync_copy(z_h.at[pl.ds(r0, npt)], acc_s.at[pl.ds(r0, npt)])
        pltpu.sync_copy(att_h, att_v)
        plsc.subcore_barrier()
        iota = lax.iota(jnp.int32, 16)
        zero16 = jnp.zeros((16,), jnp.float32)
        one0 = jnp.where(iota == 0, 1.0, 0.0).astype(jnp.float32)

        def blk(b, _):
            eb = pl.multiple_of(wid * ept + b * bsz, 8)
            pltpu.sync_copy(src_h.at[pl.ds(eb, bsz)], sidx_v)
            pltpu.sync_copy(dst_h.at[pl.ds(eb, bsz)], didx_v)
            g1 = pltpu.async_copy(xl_h.at[sidx_v], xl_v, sem1)
            g2 = pltpu.async_copy(xr_h.at[didx_v], xr_v, sem2)
            pltpu.sync_copy(me_h.at[pl.ds(eb, bsz)], me_v)
            g1.wait()
            g2.wait()

            def jloop(j, accs):
                colj = jnp.full((16,), j, jnp.int32)
                attj = plsc.load_gather(att_v, [colj])
                out = []
                for g in range(grp):
                    rows = g * 16 + iota
                    mm = (plsc.load_gather(xl_v, [rows, colj])
                          + plsc.load_gather(xr_v, [rows, colj])
                          + plsc.load_gather(me_v, [rows, colj]))
                    mm = jnp.where(mm > 0, mm, 0.2 * mm)
                    out.append(accs[g] + mm * attj)
                return tuple(out)

            accs = lax.fori_loop(0, F, jloop, (zero16,) * grp)
            for g in range(grp):
                ae = jnp.exp(accs[g])
                ae_v[...] = ae

                def eloop(ei, _):
                    r = g * 16 + ei
                    bc = plsc.load_gather(ae_v, [jnp.full((16,), ei, jnp.int32)])
                    for kk in range(F // 16):
                        con_v[r, pl.ds(kk * 16, 16)] = (
                            xl_v[r, pl.ds(kk * 16, 16)] * bc)
                    con_v[r, pl.ds(F, 16)] = bc * one0
                    return 0

                lax.fori_loop(0, 16, eloop, 0)
            pltpu.sync_copy(con_v, acc_s.at[didx_v], add=True)
            return 0

        lax.fori_loop(0, ept // bsz, blk, 0)
        plsc.subcore_barrier()
        pltpu.sync_copy(acc_s.at[pl.ds(r0, npt)], out_h.at[cid, pl.ds(r0, npt)])

    return k(xl, xr, me, src, dst, att, zeros)


def _gather_rows(tab, idx):
    kn = idx.shape[0]
    kpt = kn // NW

    @functools.partial(
        pl.kernel,
        out_type=jax.ShapeDtypeStruct((kn, 16), jnp.float32),
        mesh=_mesh(),
        compiler_params=pltpu.CompilerParams(needs_layout_passes=False, use_tc_tiling_on_sc=False),
        scratch_types=[
            pltpu.VMEM((kpt,), jnp.int32),
            pltpu.VMEM((kpt, 16), jnp.float32),
            pltpu.SemaphoreType.DMA,
        ],
    )
    def k(tab_h, idx_h, out_h, idx_v, rows_v, sem):
        cid = lax.axis_index("c")
        sid = lax.axis_index("s")
        base = pl.multiple_of((cid * NS + sid) * kpt, 8)
        pltpu.sync_copy(idx_h.at[pl.ds(base, kpt)], idx_v)
        pltpu.async_copy(tab_h.at[idx_v], rows_v, sem).wait()
        pltpu.sync_copy(rows_v, out_h.at[pl.ds(base, kpt)])

    return k(tab, idx)


def kernel(x, edge_index, edge_attr, y, node_idx,
           Wl1, bl1, Wr1, br1, We1, att1, bias1,
           Wl2, bl2, Wr2, br2, We2, att2, bias2,
           W0, b0, W1, b1, W2, b2):
    n = x.shape[0]
    npad = ((n + 8 * NS - 1) // (8 * NS)) * (8 * NS)
    src = edge_index[0].astype(jnp.int32)
    dst = edge_index[1].astype(jnp.int32)
    node_idx = node_idx.astype(jnp.int32)

    zeros_cw = jnp.zeros((npad, CW), jnp.float32)
    zeros_pw = jnp.zeros((npad, PW), jnp.float32)

    xl1, xr1 = _proj(x, Wl1, bl1, Wr1, br1)
    me1, me2 = _me2(edge_attr, We1, We2)
    pacc = _p0(dst, edge_attr, zeros_pw, npad)
    lme1, lme2 = _loopme(pacc, We1, We2, n)

    acc1 = _edge_pass(xl1, xr1, me1, src, dst, att1, zeros_cw)
    xl2, xr2 = _epi(acc1, xl1, xr1, lme1, att1, bias1, Wl2, bl2, Wr2, br2)
    acc2 = _edge_pass(xl2, xr2, me2, src, dst, att2, zeros_cw)
    outp = _head(acc2, xl2, xr2, lme2, att2, bias2, y, W0, b0, W1, b1, W2, b2)
    sel = _gather_rows(outp, node_idx)
    return sel[:, :8]


# speedup vs baseline: 4.5158x; 1.0866x over previous
import functools

import jax
import jax.numpy as jnp
from jax import lax
from jax.experimental import pallas as pl
from jax.experimental.pallas import tpu as pltpu
from jax.experimental.pallas import tpu_sc as plsc

NC = 2
NS = 16
NW = NC * NS
F = 64
CW = 80
PW = 32


def _dgt(a, b):
    return lax.dot_general(a, b, (((1,), (1,)), ((), ())),
                           preferred_element_type=jnp.float32)


def _mesh():
    return plsc.VectorSubcoreMesh(core_axis_name="c", subcore_axis_name="s")


def _proj_body(x_ref, wl_ref, bl_ref, wr_ref, br_ref, xl_ref, xr_ref):
    xb = x_ref[...]
    xl_ref[...] = _dgt(xb, wl_ref[...]) + bl_ref[...]
    xr_ref[...] = _dgt(xb, wr_ref[...]) + br_ref[...]


def _proj(x, Wl, bl, Wr, br, blk=1000):
    n, k = x.shape
    f = Wl.shape[0]
    return pl.pallas_call(
        _proj_body,
        grid=(n // blk,),
        in_specs=[
            pl.BlockSpec((blk, k), lambda i: (i, 0)),
            pl.BlockSpec((f, k), lambda i: (0, 0)),
            pl.BlockSpec((1, f), lambda i: (0, 0)),
            pl.BlockSpec((f, k), lambda i: (0, 0)),
            pl.BlockSpec((1, f), lambda i: (0, 0)),
        ],
        out_specs=[pl.BlockSpec((blk, f), lambda i: (i, 0)),
                   pl.BlockSpec((blk, f), lambda i: (i, 0))],
        out_shape=[jax.ShapeDtypeStruct((n, f), jnp.float32)] * 2,
    )(x, Wl, bl.reshape(1, -1), Wr, br.reshape(1, -1))


def _me_body(ea_ref, w1_ref, w2_ref, m1_ref, m2_ref):
    ea = ea_ref[...]
    m1_ref[...] = _dgt(ea, w1_ref[...])
    m2_ref[...] = _dgt(ea, w2_ref[...])


def _me2(ea, We1, We2, blk=4000):
    e, k = ea.shape
    f = We1.shape[0]
    return pl.pallas_call(
        _me_body,
        grid=(e // blk,),
        in_specs=[
            pl.BlockSpec((blk, k), lambda i: (i, 0)),
            pl.BlockSpec((f, k), lambda i: (0, 0)),
            pl.BlockSpec((f, k), lambda i: (0, 0)),
        ],
        out_specs=[pl.BlockSpec((blk, f), lambda i: (i, 0)),
                   pl.BlockSpec((blk, f), lambda i: (i, 0))],
        out_shape=[jax.ShapeDtypeStruct((e, f), jnp.float32)] * 2,
    )(ea, We1, We2)


def _loopme_body(pacc_ref, w1_ref, w2_ref, l1_ref, l2_ref):
    p = pacc_ref[...]
    s = p[0] + p[1]
    cnt = jnp.clip(s[:, 16:17], 1.0, None)
    la = s[:, :16] / cnt
    l1_ref[...] = _dgt(la, w1_ref[...])
    l2_ref[...] = _dgt(la, w2_ref[...])


def _loopme(pacc, We1, We2, n, blk=1000):
    f = We1.shape[0]
    return pl.pallas_call(
        _loopme_body,
        grid=(n // blk,),
        in_specs=[
            pl.BlockSpec((2, blk, PW), lambda i: (0, i, 0)),
            pl.BlockSpec((f, 16), lambda i: (0, 0)),
            pl.BlockSpec((f, 16), lambda i: (0, 0)),
        ],
        out_specs=[pl.BlockSpec((blk, f), lambda i: (i, 0)),
                   pl.BlockSpec((blk, f), lambda i: (i, 0))],
        out_shape=[jax.ShapeDtypeStruct((n, f), jnp.float32)] * 2,
    )(pacc, We1, We2)


def _node_h(acc, xl, xr, lme, att, bias):
    ms = xl + xr + lme
    ms = jnp.where(ms > 0, ms, 0.2 * ms)
    aes = jnp.exp(jnp.sum(ms * att, axis=-1, keepdims=True))
    num = acc[0, :, :F] + acc[1, :, :F] + xl * aes
    den = acc[0, :, F:F + 1] + acc[1, :, F:F + 1] + aes + 1e-16
    return jnp.maximum(num / den + bias, 0.0)


def _epi_body(acc_ref, xl_ref, xr_ref, lme_ref, att_ref, bias_ref,
              wl_ref, bl_ref, wr_ref, br_ref, xl2_ref, xr2_ref):
    h = _node_h(acc_ref[...], xl_ref[...], xr_ref[...], lme_ref[...],
                att_ref[...], bias_ref[...])
    xl2_ref[...] = _dgt(h, wl_ref[...]) + bl_ref[...]
    xr2_ref[...] = _dgt(h, wr_ref[...]) + br_ref[...]


def _epi(acc, xl, xr, lme, att, bias, Wl, bl, Wr, br, blk=1000):
    n = xl.shape[0]
    f = F
    return pl.pallas_call(
        _epi_body,
        grid=(n // blk,),
        in_specs=[
            pl.BlockSpec((2, blk, CW), lambda i: (0, i, 0)),
            pl.BlockSpec((blk, f), lambda i: (i, 0)),
            pl.BlockSpec((blk, f), lambda i: (i, 0)),
            pl.BlockSpec((blk, f), lambda i: (i, 0)),
            pl.BlockSpec((1, f), lambda i: (0, 0)),
            pl.BlockSpec((1, f), lambda i: (0, 0)),
            pl.BlockSpec((f, f), lambda i: (0, 0)),
            pl.BlockSpec((1, f), lambda i: (0, 0)),
            pl.BlockSpec((f, f), lambda i: (0, 0)),
            pl.BlockSpec((1, f), lambda i: (0, 0)),
        ],
        out_specs=[pl.BlockSpec((blk, f), lambda i: (i, 0)),
                   pl.BlockSpec((blk, f), lambda i: (i, 0))],
        out_shape=[jax.ShapeDtypeStruct((n, f), jnp.float32)] * 2,
    )(acc, xl, xr, lme, att.reshape(1, -1), bias.reshape(1, -1),
      Wl, bl.reshape(1, -1), Wr, br.reshape(1, -1))


def _head_body(acc_ref, xl_ref, xr_ref, lme_ref, att_ref, bias_ref, y_ref,
               w0_ref, b0_ref, w1a_ref, w1b_ref, b1_ref, w2_ref, b2_ref,
               out_ref):
    h2 = _node_h(acc_ref[...], xl_ref[...], xr_ref[...], lme_ref[...],
                 att_ref[...], bias_ref[...])
    y2 = jnp.maximum(_dgt(y_ref[...], w0_ref[...]) + b0_ref[...], 0.0)
    hc = jnp.maximum(_dgt(h2, w1a_ref[...]) + _dgt(y2, w1b_ref[...])
                     + b1_ref[...], 0.0)
    o = _dgt(hc, w2_ref[...]) + b2_ref[...]
    out_ref[...] = jnp.concatenate([o, jnp.zeros_like(o)], axis=1)


def _head(acc, xl, xr, lme, att, bias, y, W0, b0, W1, b1, W2, b2, blk=1000):
    n = xl.shape[0]
    f = F
    W1a = W1[:, :f]
    W1b = W1[:, f:]
    return pl.pallas_call(
        _head_body,
        grid=(n // blk,),
        in_specs=[
            pl.BlockSpec((2, blk, CW), lambda i: (0, i, 0)),
            pl.BlockSpec((blk, f), lambda i: (i, 0)),
            pl.BlockSpec((blk, f), lambda i: (i, 0)),
            pl.BlockSpec((blk, f), lambda i: (i, 0)),
            pl.BlockSpec((1, f), lambda i: (0, 0)),
            pl.BlockSpec((1, f), lambda i: (0, 0)),
            pl.BlockSpec((blk, 2), lambda i: (i, 0)),
            pl.BlockSpec((2, 2), lambda i: (0, 0)),
            pl.BlockSpec((1, 2), lambda i: (0, 0)),
            pl.BlockSpec((32, f), lambda i: (0, 0)),
            pl.BlockSpec((32, 2), lambda i: (0, 0)),
            pl.BlockSpec((1, 32), lambda i: (0, 0)),
            pl.BlockSpec((8, 32), lambda i: (0, 0)),
            pl.BlockSpec((1, 8), lambda i: (0, 0)),
        ],
        out_specs=[pl.BlockSpec((blk, 16), lambda i: (i, 0))],
        out_shape=[jax.ShapeDtypeStruct((n, 16), jnp.float32)],
    )(acc, xl, xr, lme, att.reshape(1, -1), bias.reshape(1, -1),
      y, W0, b0.reshape(1, -1), W1a, W1b, b1.reshape(1, -1),
      W2, b2.reshape(1, -1))[0]


def _p0(dst, ea, zeros, npad):
    e = dst.shape[0]
    ept = e // NW
    b0 = 400
    npt = npad // NS

    @functools.partial(
        pl.kernel,
        out_type=jax.ShapeDtypeStruct((2, npad, PW), jnp.float32),
        mesh=_mesh(),
        compiler_params=pltpu.CompilerParams(needs_layout_passes=False, use_tc_tiling_on_sc=False),
        scratch_types=[
            pltpu.VMEM((b0,), jnp.int32),
            pltpu.VMEM((b0, 16), jnp.float32),
            pltpu.VMEM((b0, PW), jnp.float32),
            pltpu.VMEM_SHARED((npad, PW), jnp.float32),
        ],
    )
    def k(dst_h, ea_h, z_h, out_h, idx_v, ea_v, con_v, acc_s):
        cid = lax.axis_index("c")
        sid = lax.axis_index("s")
        wid = cid * NS + sid
        r0 = pl.multiple_of(sid * npt, 8)
        pltpu.sync_copy(z_h.at[pl.ds(r0, npt)], acc_s.at[pl.ds(r0, npt)])
        iota = lax.iota(jnp.int32, 16)
        one0 = jnp.where(iota == 0, 1.0, 0.0).astype(jnp.float32)
        plsc.subcore_barrier()

        def blk(b, _):
            eb = pl.multiple_of(wid * ept + b * b0, 8)
            pltpu.sync_copy(dst_h.at[pl.ds(eb, b0)], idx_v)
            pltpu.sync_copy(ea_h.at[pl.ds(eb, b0)], ea_v)

            def cp(r, _):
                con_v[r, pl.ds(0, 16)] = ea_v[r, :]
                con_v[r, pl.ds(16, 16)] = one0
                return 0

            lax.fori_loop(0, b0, cp, 0)
            pltpu.sync_copy(con_v, acc_s.at[idx_v], add=True)
            return 0

        lax.fori_loop(0, ept // b0, blk, 0)
        plsc.subcore_barrier()
        pltpu.sync_copy(acc_s.at[pl.ds(r0, npt)], out_h.at[cid, pl.ds(r0, npt)])

    return k(dst, ea, zeros)


def _edge_pass(xl, xr, me, src, dst, att, zeros):
    npad = zeros.shape[0]
    e = src.shape[0]
    ept = e // NW
    bsz = 80
    grp = bsz // 16
    npt = npad // NS

    nb = ept // bsz
    assert nb % 2 == 1 and nb >= 3

    @functools.partial(
        pl.kernel,
        out_type=jax.ShapeDtypeStruct((2, npad, CW), jnp.float32),
        mesh=_mesh(),
        compiler_params=pltpu.CompilerParams(needs_layout_passes=False, use_tc_tiling_on_sc=False),
        scratch_types=[
            [pltpu.VMEM((bsz,), jnp.int32)] * 2,
            [pltpu.VMEM((bsz,), jnp.int32)] * 2,
            [pltpu.VMEM((bsz, F), jnp.float32)] * 2,
            [pltpu.VMEM((bsz, F), jnp.float32)] * 2,
            [pltpu.VMEM((bsz, F), jnp.float32)] * 2,
            [pltpu.VMEM((bsz, CW), jnp.float32)] * 2,
            pltpu.VMEM((F,), jnp.float32),
            pltpu.VMEM((16,), jnp.float32),
            pltpu.VMEM_SHARED((npad, CW), jnp.float32),
            [pltpu.SemaphoreType.DMA] * 2,
            [pltpu.SemaphoreType.DMA] * 2,
            [pltpu.SemaphoreType.DMA] * 2,
            [pltpu.SemaphoreType.DMA] * 2,
            [pltpu.SemaphoreType.DMA] * 2,
        ],
    )
    def k(xl_h, xr_h, me_h, src_h, dst_h, att_h, z_h, out_h,
          sidx, didx, xlb, xrb, meb, con, att_v, ae_v,
          acc_s, semi, semxl, semxr, semme, semsc):
        cid = lax.axis_index("c")
        sid = lax.axis_index("s")
        wid = cid * NS + sid
        r0 = pl.multiple_of(sid * npt, 8)
        pltpu.sync_copy(z_h.at[pl.ds(r0, npt)], acc_s.at[pl.ds(r0, npt)])
        pltpu.sync_copy(att_h, att_v)
        plsc.subcore_barrier()
        iota = lax.iota(jnp.int32, 16)
        zero16 = jnp.zeros((16,), jnp.float32)
        one0 = jnp.where(iota == 0, 1.0, 0.0).astype(jnp.float32)

        def ebase(bb):
            return pl.multiple_of(wid * ept + bb * bsz, 8)

        def issue_idx(s, bb):
            eb = ebase(bb)
            pltpu.async_copy(src_h.at[pl.ds(eb, bsz)], sidx[s], semi[s])
            pltpu.async_copy(dst_h.at[pl.ds(eb, bsz)], didx[s], semi[s])

        def wait_idx(s):
            pltpu.make_async_copy(src_h.at[pl.ds(0, bsz)], sidx[s], semi[s]).wait()
            pltpu.make_async_copy(dst_h.at[pl.ds(0, bsz)], didx[s], semi[s]).wait()

        def issue_gather(s, bb):
            eb = ebase(bb)
            pltpu.async_copy(xl_h.at[sidx[s]], xlb[s], semxl[s])
            pltpu.async_copy(xr_h.at[didx[s]], xrb[s], semxr[s])
            pltpu.async_copy(me_h.at[pl.ds(eb, bsz)], meb[s], semme[s])

        def wait_gather(s):
            pltpu.make_async_copy(xl_h.at[sidx[s]], xlb[s], semxl[s]).wait()
            pltpu.make_async_copy(xr_h.at[didx[s]], xrb[s], semxr[s]).wait()
            pltpu.make_async_copy(me_h.at[pl.ds(0, bsz)], meb[s], semme[s]).wait()

        def issue_scatter(s):
            pltpu.async_copy(con[s], acc_s.at[didx[s]], semsc[s], add=True)

        def wait_scatter(s):
            pltpu.make_async_copy(con[s], acc_s.at[didx[s]], semsc[s]).wait()

        def compute(s):
            xl_v, xr_v, me_v, con_v = xlb[s], xrb[s], meb[s], con[s]

            def jloop(j, accs):
                colj = jnp.full((16,), j, jnp.int32)
                attj = plsc.load_gather(att_v, [colj])
                out = []
                for g in range(grp):
                    rows = g * 16 + iota
                    mm = (plsc.load_gather(xl_v, [rows, colj])
                          + plsc.load_gather(xr_v, [rows, colj])
                          + plsc.load_gather(me_v, [rows, colj]))
                    mm = jnp.where(mm > 0, mm, 0.2 * mm)
                    out.append(accs[g] + mm * attj)
                return tuple(out)

            accs = lax.fori_loop(0, F, jloop, (zero16,) * grp)
            for g in range(grp):
                ae = jnp.exp(accs[g])
                ae_v[...] = ae

                def eloop(ei, _):
                    r = g * 16 + ei
                    bc = plsc.load_gather(ae_v, [jnp.full((16,), ei, jnp.int32)])
                    for kk in range(F // 16):
                        con_v[r, pl.ds(kk * 16, 16)] = (
                            xl_v[r, pl.ds(kk * 16, 16)] * bc)
                    con_v[r, pl.ds(F, 16)] = bc * one0
                    return 0

                lax.fori_loop(0, 16, eloop, 0)

        pltpu.sync_copy(src_h.at[pl.ds(ebase(0), bsz)], sidx[0])
        pltpu.sync_copy(dst_h.at[pl.ds(ebase(0), bsz)], didx[0])
        issue_gather(0, 0)
        issue_idx(1, 1)
        wait_gather(0)
        compute(0)
        issue_scatter(0)
        wait_idx(1)
        issue_gather(1, 1)

        def pair(i, _):
            bb = 2 * i + 2
            wait_scatter(0)
            issue_idx(0, bb)
            wait_gather(1)
            compute(1)
            issue_scatter(1)
            wait_idx(0)
            issue_gather(0, bb)
            wait_scatter(1)

            @pl.when(bb + 1 < nb)
            def _():
                issue_idx(1, bb + 1)

            wait_gather(0)
            compute(0)
            issue_scatter(0)

            @pl.when(bb + 1 < nb)
            def _():
                wait_idx(1)
                issue_gather(1, bb + 1)

            return 0

        lax.fori_loop(0, (nb - 1) // 2, pair, 0)
        wait_scatter(0)
        plsc.subcore_barrier()
        pltpu.sync_copy(acc_s.at[pl.ds(r0, npt)], out_h.at[cid, pl.ds(r0, npt)])

    return k(xl, xr, me, src, dst, att, zeros)


def _gather_rows(tab, idx):
    kn = idx.shape[0]
    kpt = kn // NW

    @functools.partial(
        pl.kernel,
        out_type=jax.ShapeDtypeStruct((kn, 16), jnp.float32),
        mesh=_mesh(),
        compiler_params=pltpu.CompilerParams(needs_layout_passes=False, use_tc_tiling_on_sc=False),
        scratch_types=[
            pltpu.VMEM((kpt,), jnp.int32),
            pltpu.VMEM((kpt, 16), jnp.float32),
            pltpu.SemaphoreType.DMA,
        ],
    )
    def k(tab_h, idx_h, out_h, idx_v, rows_v, sem):
        cid = lax.axis_index("c")
        sid = lax.axis_index("s")
        base = pl.multiple_of((cid * NS + sid) * kpt, 8)
        pltpu.sync_copy(idx_h.at[pl.ds(base, kpt)], idx_v)
        pltpu.async_copy(tab_h.at[idx_v], rows_v, sem).wait()
        pltpu.sync_copy(rows_v, out_h.at[pl.ds(base, kpt)])

    return k(tab, idx)


def kernel(x, edge_index, edge_attr, y, node_idx,
           Wl1, bl1, Wr1, br1, We1, att1, bias1,
           Wl2, bl2, Wr2, br2, We2, att2, bias2,
           W0, b0, W1, b1, W2, b2):
    n = x.shape[0]
    npad = ((n + 8 * NS - 1) // (8 * NS)) * (8 * NS)
    src = edge_index[0].astype(jnp.int32)
    dst = edge_index[1].astype(jnp.int32)
    node_idx = node_idx.astype(jnp.int32)

    zeros_cw = jnp.zeros((npad, CW), jnp.float32)
    zeros_pw = jnp.zeros((npad, PW), jnp.float32)

    xl1, xr1 = _proj(x, Wl1, bl1, Wr1, br1)
    me1, me2 = _me2(edge_attr, We1, We2)
    pacc = _p0(dst, edge_attr, zeros_pw, npad)
    lme1, lme2 = _loopme(pacc, We1, We2, n)

    acc1 = _edge_pass(xl1, xr1, me1, src, dst, att1, zeros_cw)
    xl2, xr2 = _epi(acc1, xl1, xr1, lme1, att1, bias1, Wl2, bl2, Wr2, br2)
    acc2 = _edge_pass(xl2, xr2, me2, src, dst, att2, zeros_cw)
    outp = _head(acc2, xl2, xr2, lme2, att2, bias2, y, W0, b0, W1, b1, W2, b2)
    sel = _gather_rows(outp, node_idx)
    return sel[:, :8]


# speedup vs baseline: 9.8761x; 2.1870x over previous
import functools

import jax
import jax.numpy as jnp
from jax import lax
from jax.experimental import pallas as pl
from jax.experimental.pallas import tpu as pltpu
from jax.experimental.pallas import tpu_sc as plsc

NC = 2
NS = 16
NW = NC * NS
F = 64
CW = 80
PW = 32


def _dgt(a, b):
    return lax.dot_general(a, b, (((1,), (1,)), ((), ())),
                           preferred_element_type=jnp.float32)


def _mesh():
    return plsc.VectorSubcoreMesh(core_axis_name="c", subcore_axis_name="s")


def _proj_body(x_ref, wl_ref, bl_ref, wr_ref, br_ref, xl_ref, xr_ref):
    xb = x_ref[...]
    xl_ref[...] = _dgt(xb, wl_ref[...]) + bl_ref[...]
    xr_ref[...] = _dgt(xb, wr_ref[...]) + br_ref[...]


def _proj(x, Wl, bl, Wr, br, blk=1000):
    n, k = x.shape
    f = Wl.shape[0]
    return pl.pallas_call(
        _proj_body,
        grid=(n // blk,),
        in_specs=[
            pl.BlockSpec((blk, k), lambda i: (i, 0)),
            pl.BlockSpec((f, k), lambda i: (0, 0)),
            pl.BlockSpec((1, f), lambda i: (0, 0)),
            pl.BlockSpec((f, k), lambda i: (0, 0)),
            pl.BlockSpec((1, f), lambda i: (0, 0)),
        ],
        out_specs=[pl.BlockSpec((blk, f), lambda i: (i, 0)),
                   pl.BlockSpec((blk, f), lambda i: (i, 0))],
        out_shape=[jax.ShapeDtypeStruct((n, f), jnp.float32)] * 2,
    )(x, Wl, bl.reshape(1, -1), Wr, br.reshape(1, -1))


def _me_body(ea_ref, w1_ref, w2_ref, m1_ref, m2_ref):
    ea = ea_ref[...]
    m1_ref[...] = _dgt(ea, w1_ref[...])
    m2_ref[...] = _dgt(ea, w2_ref[...])


def _me2(ea, We1, We2, blk=4000):
    e, k = ea.shape
    f = We1.shape[0]
    return pl.pallas_call(
        _me_body,
        grid=(e // blk,),
        in_specs=[
            pl.BlockSpec((blk, k), lambda i: (i, 0)),
            pl.BlockSpec((f, k), lambda i: (0, 0)),
            pl.BlockSpec((f, k), lambda i: (0, 0)),
        ],
        out_specs=[pl.BlockSpec((blk, f), lambda i: (i, 0)),
                   pl.BlockSpec((blk, f), lambda i: (i, 0))],
        out_shape=[jax.ShapeDtypeStruct((e, f), jnp.float32)] * 2,
    )(ea, We1, We2)


def _loopme_body(pacc_ref, w1_ref, w2_ref, l1_ref, l2_ref):
    p = pacc_ref[...]
    s = p[0] + p[1]
    cnt = jnp.clip(s[:, 16:17], 1.0, None)
    la = s[:, :16] / cnt
    l1_ref[...] = _dgt(la, w1_ref[...])
    l2_ref[...] = _dgt(la, w2_ref[...])


def _loopme(pacc, We1, We2, n, blk=1000):
    f = We1.shape[0]
    return pl.pallas_call(
        _loopme_body,
        grid=(n // blk,),
        in_specs=[
            pl.BlockSpec((2, blk, PW), lambda i: (0, i, 0)),
            pl.BlockSpec((f, 16), lambda i: (0, 0)),
            pl.BlockSpec((f, 16), lambda i: (0, 0)),
        ],
        out_specs=[pl.BlockSpec((blk, f), lambda i: (i, 0)),
                   pl.BlockSpec((blk, f), lambda i: (i, 0))],
        out_shape=[jax.ShapeDtypeStruct((n, f), jnp.float32)] * 2,
    )(pacc, We1, We2)


def _node_h(acc, xl, xr, lme, att, bias):
    ms = xl + xr + lme
    ms = jnp.where(ms > 0, ms, 0.2 * ms)
    aes = jnp.exp(jnp.sum(ms * att, axis=-1, keepdims=True))
    num = acc[0, :, :F] + acc[1, :, :F] + xl * aes
    den = acc[0, :, F:F + 1] + acc[1, :, F:F + 1] + aes + 1e-16
    return jnp.maximum(num / den + bias, 0.0)


def _epi_body(acc_ref, xl_ref, xr_ref, lme_ref, att_ref, bias_ref,
              wl_ref, bl_ref, wr_ref, br_ref, xl2_ref, xr2_ref):
    h = _node_h(acc_ref[...], xl_ref[...], xr_ref[...], lme_ref[...],
                att_ref[...], bias_ref[...])
    xl2_ref[...] = _dgt(h, wl_ref[...]) + bl_ref[...]
    xr2_ref[...] = _dgt(h, wr_ref[...]) + br_ref[...]


def _epi(acc, xl, xr, lme, att, bias, Wl, bl, Wr, br, blk=1000):
    n = xl.shape[0]
    f = F
    return pl.pallas_call(
        _epi_body,
        grid=(n // blk,),
        in_specs=[
            pl.BlockSpec((2, blk, CW), lambda i: (0, i, 0)),
            pl.BlockSpec((blk, f), lambda i: (i, 0)),
            pl.BlockSpec((blk, f), lambda i: (i, 0)),
            pl.BlockSpec((blk, f), lambda i: (i, 0)),
            pl.BlockSpec((1, f), lambda i: (0, 0)),
            pl.BlockSpec((1, f), lambda i: (0, 0)),
            pl.BlockSpec((f, f), lambda i: (0, 0)),
            pl.BlockSpec((1, f), lambda i: (0, 0)),
            pl.BlockSpec((f, f), lambda i: (0, 0)),
            pl.BlockSpec((1, f), lambda i: (0, 0)),
        ],
        out_specs=[pl.BlockSpec((blk, f), lambda i: (i, 0)),
                   pl.BlockSpec((blk, f), lambda i: (i, 0))],
        out_shape=[jax.ShapeDtypeStruct((n, f), jnp.float32)] * 2,
    )(acc, xl, xr, lme, att.reshape(1, -1), bias.reshape(1, -1),
      Wl, bl.reshape(1, -1), Wr, br.reshape(1, -1))


def _head_body(acc_ref, xl_ref, xr_ref, lme_ref, att_ref, bias_ref, y_ref,
               w0_ref, b0_ref, w1a_ref, w1b_ref, b1_ref, w2_ref, b2_ref,
               out_ref):
    h2 = _node_h(acc_ref[...], xl_ref[...], xr_ref[...], lme_ref[...],
                 att_ref[...], bias_ref[...])
    y2 = jnp.maximum(_dgt(y_ref[...], w0_ref[...]) + b0_ref[...], 0.0)
    hc = jnp.maximum(_dgt(h2, w1a_ref[...]) + _dgt(y2, w1b_ref[...])
                     + b1_ref[...], 0.0)
    o = _dgt(hc, w2_ref[...]) + b2_ref[...]
    out_ref[...] = jnp.concatenate([o, jnp.zeros_like(o)], axis=1)


def _head(acc, xl, xr, lme, att, bias, y, W0, b0, W1, b1, W2, b2, blk=1000):
    n = xl.shape[0]
    f = F
    W1a = W1[:, :f]
    W1b = W1[:, f:]
    return pl.pallas_call(
        _head_body,
        grid=(n // blk,),
        in_specs=[
            pl.BlockSpec((2, blk, CW), lambda i: (0, i, 0)),
            pl.BlockSpec((blk, f), lambda i: (i, 0)),
            pl.BlockSpec((blk, f), lambda i: (i, 0)),
            pl.BlockSpec((blk, f), lambda i: (i, 0)),
            pl.BlockSpec((1, f), lambda i: (0, 0)),
            pl.BlockSpec((1, f), lambda i: (0, 0)),
            pl.BlockSpec((blk, 2), lambda i: (i, 0)),
            pl.BlockSpec((2, 2), lambda i: (0, 0)),
            pl.BlockSpec((1, 2), lambda i: (0, 0)),
            pl.BlockSpec((32, f), lambda i: (0, 0)),
            pl.BlockSpec((32, 2), lambda i: (0, 0)),
            pl.BlockSpec((1, 32), lambda i: (0, 0)),
            pl.BlockSpec((8, 32), lambda i: (0, 0)),
            pl.BlockSpec((1, 8), lambda i: (0, 0)),
        ],
        out_specs=[pl.BlockSpec((blk, 16), lambda i: (i, 0))],
        out_shape=[jax.ShapeDtypeStruct((n, 16), jnp.float32)],
    )(acc, xl, xr, lme, att.reshape(1, -1), bias.reshape(1, -1),
      y, W0, b0.reshape(1, -1), W1a, W1b, b1.reshape(1, -1),
      W2, b2.reshape(1, -1))[0]


def _p0(dst, ea, zeros, npad):
    e = dst.shape[0]
    ept = e // NW
    b0 = 400
    npt = npad // NS

    @functools.partial(
        pl.kernel,
        out_type=jax.ShapeDtypeStruct((2, npad, PW), jnp.float32),
        mesh=_mesh(),
        compiler_params=pltpu.CompilerParams(needs_layout_passes=False, use_tc_tiling_on_sc=False),
        scratch_types=[
            pltpu.VMEM((b0,), jnp.int32),
            pltpu.VMEM((b0, 16), jnp.float32),
            pltpu.VMEM((b0, PW), jnp.float32),
            pltpu.VMEM_SHARED((npad, PW), jnp.float32),
        ],
    )
    def k(dst_h, ea_h, z_h, out_h, idx_v, ea_v, con_v, acc_s):
        cid = lax.axis_index("c")
        sid = lax.axis_index("s")
        wid = cid * NS + sid
        r0 = pl.multiple_of(sid * npt, 8)
        pltpu.sync_copy(z_h.at[pl.ds(r0, npt)], acc_s.at[pl.ds(r0, npt)])
        iota = lax.iota(jnp.int32, 16)
        one0 = jnp.where(iota == 0, 1.0, 0.0).astype(jnp.float32)
        plsc.subcore_barrier()

        def blk(b, _):
            eb = pl.multiple_of(wid * ept + b * b0, 8)
            pltpu.sync_copy(dst_h.at[pl.ds(eb, b0)], idx_v)
            pltpu.sync_copy(ea_h.at[pl.ds(eb, b0)], ea_v)

            def cp(r, _):
                con_v[r, pl.ds(0, 16)] = ea_v[r, :]
                con_v[r, pl.ds(16, 16)] = one0
                return 0

            lax.fori_loop(0, b0, cp, 0)
            pltpu.sync_copy(con_v, acc_s.at[idx_v], add=True)
            return 0

        lax.fori_loop(0, ept // b0, blk, 0)
        plsc.subcore_barrier()
        pltpu.sync_copy(acc_s.at[pl.ds(r0, npt)], out_h.at[cid, pl.ds(r0, npt)])

    return k(dst, ea, zeros)


def _edge_pass(xl, xr, me, src, dst, att, zeros):
    npad = zeros.shape[0]
    e = src.shape[0]
    ept = e // NW
    bsz = 80
    grp = bsz // 16
    npt = npad // NS

    nb = ept // bsz
    assert nb % 2 == 1 and nb >= 3

    @functools.partial(
        pl.kernel,
        out_type=jax.ShapeDtypeStruct((2, npad, CW), jnp.float32),
        mesh=_mesh(),
        compiler_params=pltpu.CompilerParams(needs_layout_passes=False, use_tc_tiling_on_sc=False),
        scratch_types=[
            [pltpu.VMEM((bsz,), jnp.int32)] * 2,
            [pltpu.VMEM((bsz,), jnp.int32)] * 2,
            [pltpu.VMEM((bsz, F), jnp.float32)] * 2,
            [pltpu.VMEM((bsz, F), jnp.float32)] * 2,
            [pltpu.VMEM((bsz, F), jnp.float32)] * 2,
            [pltpu.VMEM((bsz, CW), jnp.float32)] * 2,
            pltpu.VMEM((F,), jnp.float32),
            pltpu.VMEM((16,), jnp.float32),
            pltpu.VMEM_SHARED((npad, CW), jnp.float32),
            [pltpu.SemaphoreType.DMA] * 2,
            [pltpu.SemaphoreType.DMA] * 2,
            [pltpu.SemaphoreType.DMA] * 2,
            [pltpu.SemaphoreType.DMA] * 2,
            [pltpu.SemaphoreType.DMA] * 2,
        ],
    )
    def k(xl_h, xr_h, me_h, src_h, dst_h, att_h, z_h, out_h,
          sidx, didx, xlb, xrb, meb, con, att_v, ae_v,
          acc_s, semi, semxl, semxr, semme, semsc):
        cid = lax.axis_index("c")
        sid = lax.axis_index("s")
        wid = cid * NS + sid
        r0 = pl.multiple_of(sid * npt, 8)
        pltpu.sync_copy(z_h.at[pl.ds(r0, npt)], acc_s.at[pl.ds(r0, npt)])
        pltpu.sync_copy(att_h, att_v)
        plsc.subcore_barrier()
        iota = lax.iota(jnp.int32, 16)
        zero16 = jnp.zeros((16,), jnp.float32)
        one0 = jnp.where(iota == 0, 1.0, 0.0).astype(jnp.float32)

        def ebase(bb):
            return pl.multiple_of(wid * ept + bb * bsz, 8)

        def issue_idx(s, bb):
            eb = ebase(bb)
            pltpu.async_copy(src_h.at[pl.ds(eb, bsz)], sidx[s], semi[s])
            pltpu.async_copy(dst_h.at[pl.ds(eb, bsz)], didx[s], semi[s])

        def wait_idx(s):
            pltpu.make_async_copy(src_h.at[pl.ds(0, bsz)], sidx[s], semi[s]).wait()
            pltpu.make_async_copy(dst_h.at[pl.ds(0, bsz)], didx[s], semi[s]).wait()

        def issue_gather(s, bb):
            eb = ebase(bb)
            pltpu.async_copy(xl_h.at[sidx[s]], xlb[s], semxl[s])
            pltpu.async_copy(xr_h.at[didx[s]], xrb[s], semxr[s])
            pltpu.async_copy(me_h.at[pl.ds(eb, bsz)], meb[s], semme[s])

        def wait_gather(s):
            pltpu.make_async_copy(xl_h.at[sidx[s]], xlb[s], semxl[s]).wait()
            pltpu.make_async_copy(xr_h.at[didx[s]], xrb[s], semxr[s]).wait()
            pltpu.make_async_copy(me_h.at[pl.ds(0, bsz)], meb[s], semme[s]).wait()

        def issue_scatter(s):
            pltpu.async_copy(con[s], acc_s.at[didx[s]], semsc[s], add=True)

        def wait_scatter(s):
            pltpu.make_async_copy(con[s], acc_s.at[didx[s]], semsc[s]).wait()

        def compute(s):
            xl_v, xr_v, me_v, con_v = xlb[s], xrb[s], meb[s], con[s]
            nch = F // 16
            att_c = [att_v[pl.ds(kk * 16, 16)] for kk in range(nch)]
            unroll = 4

            def eloop(it, _):
                for u in range(unroll):
                    r = it * unroll + u
                    xs = [xl_v[r, pl.ds(kk * 16, 16)] for kk in range(nch)]
                    acc = zero16
                    for kk in range(nch):
                        mm = (xs[kk] + xr_v[r, pl.ds(kk * 16, 16)]
                              + me_v[r, pl.ds(kk * 16, 16)])
                        mm = jnp.where(mm > 0, mm, 0.2 * mm)
                        acc = acc + mm * att_c[kk]
                    alpha = jnp.sum(acc)
                    bc = jnp.exp(jnp.broadcast_to(alpha, (16,)))
                    for kk in range(nch):
                        con_v[r, pl.ds(kk * 16, 16)] = xs[kk] * bc
                    con_v[r, pl.ds(F, 16)] = bc * one0
                return 0

            lax.fori_loop(0, bsz // unroll, eloop, 0)

        pltpu.sync_copy(src_h.at[pl.ds(ebase(0), bsz)], sidx[0])
        pltpu.sync_copy(dst_h.at[pl.ds(ebase(0), bsz)], didx[0])
        issue_gather(0, 0)
        issue_idx(1, 1)
        wait_gather(0)
        compute(0)
        issue_scatter(0)
        wait_idx(1)
        issue_gather(1, 1)

        def pair(i, _):
            bb = 2 * i + 2
            wait_scatter(0)
            issue_idx(0, bb)
            wait_gather(1)
            compute(1)
            issue_scatter(1)
            wait_idx(0)
            issue_gather(0, bb)
            wait_scatter(1)

            @pl.when(bb + 1 < nb)
            def _():
                issue_idx(1, bb + 1)

            wait_gather(0)
            compute(0)
            issue_scatter(0)

            @pl.when(bb + 1 < nb)
            def _():
                wait_idx(1)
                issue_gather(1, bb + 1)

            return 0

        lax.fori_loop(0, (nb - 1) // 2, pair, 0)
        wait_scatter(0)
        plsc.subcore_barrier()
        pltpu.sync_copy(acc_s.at[pl.ds(r0, npt)], out_h.at[cid, pl.ds(r0, npt)])

    return k(xl, xr, me, src, dst, att, zeros)


def _gather_rows(tab, idx):
    kn = idx.shape[0]
    kpt = kn // NW

    @functools.partial(
        pl.kernel,
        out_type=jax.ShapeDtypeStruct((kn, 16), jnp.float32),
        mesh=_mesh(),
        compiler_params=pltpu.CompilerParams(needs_layout_passes=False, use_tc_tiling_on_sc=False),
        scratch_types=[
            pltpu.VMEM((kpt,), jnp.int32),
            pltpu.VMEM((kpt, 16), jnp.float32),
            pltpu.SemaphoreType.DMA,
        ],
    )
    def k(tab_h, idx_h, out_h, idx_v, rows_v, sem):
        cid = lax.axis_index("c")
        sid = lax.axis_index("s")
        base = pl.multiple_of((cid * NS + sid) * kpt, 8)
        pltpu.sync_copy(idx_h.at[pl.ds(base, kpt)], idx_v)
        pltpu.async_copy(tab_h.at[idx_v], rows_v, sem).wait()
        pltpu.sync_copy(rows_v, out_h.at[pl.ds(base, kpt)])

    return k(tab, idx)


def kernel(x, edge_index, edge_attr, y, node_idx,
           Wl1, bl1, Wr1, br1, We1, att1, bias1,
           Wl2, bl2, Wr2, br2, We2, att2, bias2,
           W0, b0, W1, b1, W2, b2):
    n = x.shape[0]
    npad = ((n + 8 * NS - 1) // (8 * NS)) * (8 * NS)
    src = edge_index[0].astype(jnp.int32)
    dst = edge_index[1].astype(jnp.int32)
    node_idx = node_idx.astype(jnp.int32)

    zeros_cw = jnp.zeros((npad, CW), jnp.float32)
    zeros_pw = jnp.zeros((npad, PW), jnp.float32)

    xl1, xr1 = _proj(x, Wl1, bl1, Wr1, br1)
    me1, me2 = _me2(edge_attr, We1, We2)
    pacc = _p0(dst, edge_attr, zeros_pw, npad)
    lme1, lme2 = _loopme(pacc, We1, We2, n)

    acc1 = _edge_pass(xl1, xr1, me1, src, dst, att1, zeros_cw)
    xl2, xr2 = _epi(acc1, xl1, xr1, lme1, att1, bias1, Wl2, bl2, Wr2, br2)
    acc2 = _edge_pass(xl2, xr2, me2, src, dst, att2, zeros_cw)
    outp = _head(acc2, xl2, xr2, lme2, att2, bias2, y, W0, b0, W1, b1, W2, b2)
    sel = _gather_rows(outp, node_idx)
    return sel[:, :8]


# speedup vs baseline: 16.2926x; 1.6497x over previous
import functools

import jax
import jax.numpy as jnp
from jax import lax
from jax.experimental import pallas as pl
from jax.experimental.pallas import tpu as pltpu
from jax.experimental.pallas import tpu_sc as plsc

NC = 2
NS = 16
NW = NC * NS
F = 64
CW = 80
PW = 32


def _dgt(a, b):
    return lax.dot_general(a, b, (((1,), (1,)), ((), ())),
                           preferred_element_type=jnp.float32)


def _mesh():
    return plsc.VectorSubcoreMesh(core_axis_name="c", subcore_axis_name="s")


def _proj_body(x_ref, wl_ref, bl_ref, wr_ref, br_ref, xl_ref, xr_ref):
    xb = x_ref[...]
    xl_ref[...] = _dgt(xb, wl_ref[...]) + bl_ref[...]
    xr_ref[...] = _dgt(xb, wr_ref[...]) + br_ref[...]


def _proj(x, Wl, bl, Wr, br, blk=1000):
    n, k = x.shape
    f = Wl.shape[0]
    return pl.pallas_call(
        _proj_body,
        grid=(n // blk,),
        in_specs=[
            pl.BlockSpec((blk, k), lambda i: (i, 0)),
            pl.BlockSpec((f, k), lambda i: (0, 0)),
            pl.BlockSpec((1, f), lambda i: (0, 0)),
            pl.BlockSpec((f, k), lambda i: (0, 0)),
            pl.BlockSpec((1, f), lambda i: (0, 0)),
        ],
        out_specs=[pl.BlockSpec((blk, f), lambda i: (i, 0)),
                   pl.BlockSpec((blk, f), lambda i: (i, 0))],
        out_shape=[jax.ShapeDtypeStruct((n, f), jnp.float32)] * 2,
    )(x, Wl, bl.reshape(1, -1), Wr, br.reshape(1, -1))


def _me_body(ea_ref, w1_ref, w2_ref, m1_ref, m2_ref):
    ea = ea_ref[...]
    m1_ref[...] = _dgt(ea, w1_ref[...])
    m2_ref[...] = _dgt(ea, w2_ref[...])


def _me2(ea, We1, We2, blk=4000):
    e, k = ea.shape
    f = We1.shape[0]
    return pl.pallas_call(
        _me_body,
        grid=(e // blk,),
        in_specs=[
            pl.BlockSpec((blk, k), lambda i: (i, 0)),
            pl.BlockSpec((f, k), lambda i: (0, 0)),
            pl.BlockSpec((f, k), lambda i: (0, 0)),
        ],
        out_specs=[pl.BlockSpec((blk, f), lambda i: (i, 0)),
                   pl.BlockSpec((blk, f), lambda i: (i, 0))],
        out_shape=[jax.ShapeDtypeStruct((e, f), jnp.float32)] * 2,
    )(ea, We1, We2)


def _loopme_body(pacc_ref, w1_ref, w2_ref, l1_ref, l2_ref):
    p = pacc_ref[...]
    s = p[0] + p[1]
    cnt = jnp.clip(s[:, 16:17], 1.0, None)
    la = s[:, :16] / cnt
    l1_ref[...] = _dgt(la, w1_ref[...])
    l2_ref[...] = _dgt(la, w2_ref[...])


def _loopme(pacc, We1, We2, n, blk=1000):
    f = We1.shape[0]
    return pl.pallas_call(
        _loopme_body,
        grid=(n // blk,),
        in_specs=[
            pl.BlockSpec((2, blk, PW), lambda i: (0, i, 0)),
            pl.BlockSpec((f, 16), lambda i: (0, 0)),
            pl.BlockSpec((f, 16), lambda i: (0, 0)),
        ],
        out_specs=[pl.BlockSpec((blk, f), lambda i: (i, 0)),
                   pl.BlockSpec((blk, f), lambda i: (i, 0))],
        out_shape=[jax.ShapeDtypeStruct((n, f), jnp.float32)] * 2,
    )(pacc, We1, We2)


def _node_h(acc, xl, xr, lme, att, bias):
    ms = xl + xr + lme
    ms = jnp.where(ms > 0, ms, 0.2 * ms)
    aes = jnp.exp(jnp.sum(ms * att, axis=-1, keepdims=True))
    num = acc[0, :, :F] + acc[1, :, :F] + xl * aes
    den = acc[0, :, F:F + 1] + acc[1, :, F:F + 1] + aes + 1e-16
    return jnp.maximum(num / den + bias, 0.0)


def _epi_body(acc_ref, xl_ref, xr_ref, lme_ref, att_ref, bias_ref,
              wl_ref, bl_ref, wr_ref, br_ref, xl2_ref, xr2_ref):
    h = _node_h(acc_ref[...], xl_ref[...], xr_ref[...], lme_ref[...],
                att_ref[...], bias_ref[...])
    xl2_ref[...] = _dgt(h, wl_ref[...]) + bl_ref[...]
    xr2_ref[...] = _dgt(h, wr_ref[...]) + br_ref[...]


def _epi(acc, xl, xr, lme, att, bias, Wl, bl, Wr, br, blk=1000):
    n = xl.shape[0]
    f = F
    return pl.pallas_call(
        _epi_body,
        grid=(n // blk,),
        in_specs=[
            pl.BlockSpec((2, blk, CW), lambda i: (0, i, 0)),
            pl.BlockSpec((blk, f), lambda i: (i, 0)),
            pl.BlockSpec((blk, f), lambda i: (i, 0)),
            pl.BlockSpec((blk, f), lambda i: (i, 0)),
            pl.BlockSpec((1, f), lambda i: (0, 0)),
            pl.BlockSpec((1, f), lambda i: (0, 0)),
            pl.BlockSpec((f, f), lambda i: (0, 0)),
            pl.BlockSpec((1, f), lambda i: (0, 0)),
            pl.BlockSpec((f, f), lambda i: (0, 0)),
            pl.BlockSpec((1, f), lambda i: (0, 0)),
        ],
        out_specs=[pl.BlockSpec((blk, f), lambda i: (i, 0)),
                   pl.BlockSpec((blk, f), lambda i: (i, 0))],
        out_shape=[jax.ShapeDtypeStruct((n, f), jnp.float32)] * 2,
    )(acc, xl, xr, lme, att.reshape(1, -1), bias.reshape(1, -1),
      Wl, bl.reshape(1, -1), Wr, br.reshape(1, -1))


def _head_body(acc_ref, xl_ref, xr_ref, lme_ref, att_ref, bias_ref, y_ref,
               w0_ref, b0_ref, w1a_ref, w1b_ref, b1_ref, w2_ref, b2_ref,
               out_ref):
    h2 = _node_h(acc_ref[...], xl_ref[...], xr_ref[...], lme_ref[...],
                 att_ref[...], bias_ref[...])
    y2 = jnp.maximum(_dgt(y_ref[...], w0_ref[...]) + b0_ref[...], 0.0)
    hc = jnp.maximum(_dgt(h2, w1a_ref[...]) + _dgt(y2, w1b_ref[...])
                     + b1_ref[...], 0.0)
    o = _dgt(hc, w2_ref[...]) + b2_ref[...]
    out_ref[...] = jnp.concatenate([o, jnp.zeros_like(o)], axis=1)


def _head(acc, xl, xr, lme, att, bias, y, W0, b0, W1, b1, W2, b2, blk=1000):
    n = xl.shape[0]
    f = F
    W1a = W1[:, :f]
    W1b = W1[:, f:]
    return pl.pallas_call(
        _head_body,
        grid=(n // blk,),
        in_specs=[
            pl.BlockSpec((2, blk, CW), lambda i: (0, i, 0)),
            pl.BlockSpec((blk, f), lambda i: (i, 0)),
            pl.BlockSpec((blk, f), lambda i: (i, 0)),
            pl.BlockSpec((blk, f), lambda i: (i, 0)),
            pl.BlockSpec((1, f), lambda i: (0, 0)),
            pl.BlockSpec((1, f), lambda i: (0, 0)),
            pl.BlockSpec((blk, 2), lambda i: (i, 0)),
            pl.BlockSpec((2, 2), lambda i: (0, 0)),
            pl.BlockSpec((1, 2), lambda i: (0, 0)),
            pl.BlockSpec((32, f), lambda i: (0, 0)),
            pl.BlockSpec((32, 2), lambda i: (0, 0)),
            pl.BlockSpec((1, 32), lambda i: (0, 0)),
            pl.BlockSpec((8, 32), lambda i: (0, 0)),
            pl.BlockSpec((1, 8), lambda i: (0, 0)),
        ],
        out_specs=[pl.BlockSpec((blk, 16), lambda i: (i, 0))],
        out_shape=[jax.ShapeDtypeStruct((n, 16), jnp.float32)],
    )(acc, xl, xr, lme, att.reshape(1, -1), bias.reshape(1, -1),
      y, W0, b0.reshape(1, -1), W1a, W1b, b1.reshape(1, -1),
      W2, b2.reshape(1, -1))[0]


def _p0(dst, ea, zeros, npad):
    e = dst.shape[0]
    ept = e // NW
    b0 = 400
    npt = npad // NS

    @functools.partial(
        pl.kernel,
        out_type=jax.ShapeDtypeStruct((2, npad, PW), jnp.float32),
        mesh=_mesh(),
        compiler_params=pltpu.CompilerParams(needs_layout_passes=False, use_tc_tiling_on_sc=False),
        scratch_types=[
            pltpu.VMEM((b0,), jnp.int32),
            pltpu.VMEM((b0, 16), jnp.float32),
            pltpu.VMEM((b0, PW), jnp.float32),
            pltpu.VMEM_SHARED((npad, PW), jnp.float32),
        ],
    )
    def k(dst_h, ea_h, z_h, out_h, idx_v, ea_v, con_v, acc_s):
        cid = lax.axis_index("c")
        sid = lax.axis_index("s")
        wid = cid * NS + sid
        r0 = pl.multiple_of(sid * npt, 8)
        pltpu.sync_copy(z_h.at[pl.ds(r0, npt)], acc_s.at[pl.ds(r0, npt)])
        iota = lax.iota(jnp.int32, 16)
        one0 = jnp.where(iota == 0, 1.0, 0.0).astype(jnp.float32)
        plsc.subcore_barrier()

        def blk(b, _):
            eb = pl.multiple_of(wid * ept + b * b0, 8)
            pltpu.sync_copy(dst_h.at[pl.ds(eb, b0)], idx_v)
            pltpu.sync_copy(ea_h.at[pl.ds(eb, b0)], ea_v)

            def cp(r, _):
                con_v[r, pl.ds(0, 16)] = ea_v[r, :]
                con_v[r, pl.ds(16, 16)] = one0
                return 0

            lax.fori_loop(0, b0, cp, 0)
            pltpu.sync_copy(con_v, acc_s.at[idx_v], add=True)
            return 0

        lax.fori_loop(0, ept // b0, blk, 0)
        plsc.subcore_barrier()
        pltpu.sync_copy(acc_s.at[pl.ds(r0, npt)], out_h.at[cid, pl.ds(r0, npt)])

    return k(dst, ea, zeros)


def _edge_pass(xl, xr, me, src, dst, att, zeros):
    npad = zeros.shape[0]
    e = src.shape[0]
    ept = e // NW
    bsz = 80
    grp = bsz // 16
    npt = npad // NS

    nb = ept // bsz
    assert nb % 2 == 1 and nb >= 3

    @functools.partial(
        pl.kernel,
        out_type=jax.ShapeDtypeStruct((2, npad, CW), jnp.float32),
        mesh=_mesh(),
        compiler_params=pltpu.CompilerParams(needs_layout_passes=False, use_tc_tiling_on_sc=False),
        scratch_types=[
            [pltpu.VMEM((bsz,), jnp.int32)] * 2,
            [pltpu.VMEM((bsz,), jnp.int32)] * 2,
            [pltpu.VMEM((bsz, F), jnp.float32)] * 2,
            [pltpu.VMEM((bsz, F), jnp.float32)] * 2,
            [pltpu.VMEM((bsz, F), jnp.float32)] * 2,
            [pltpu.VMEM((bsz, CW), jnp.float32)] * 2,
            pltpu.VMEM((F,), jnp.float32),
            pltpu.VMEM((16,), jnp.float32),
            pltpu.VMEM_SHARED((npad, CW), jnp.float32),
            [pltpu.SemaphoreType.DMA] * 2,
            [pltpu.SemaphoreType.DMA] * 2,
            [pltpu.SemaphoreType.DMA] * 2,
            [pltpu.SemaphoreType.DMA] * 2,
            [pltpu.SemaphoreType.DMA] * 2,
        ],
    )
    def k(xl_h, xr_h, me_h, src_h, dst_h, att_h, z_h, out_h,
          sidx, didx, xlb, xrb, meb, con, att_v, ae_v,
          acc_s, semi, semxl, semxr, semme, semsc):
        cid = lax.axis_index("c")
        sid = lax.axis_index("s")
        wid = cid * NS + sid
        r0 = pl.multiple_of(sid * npt, 8)
        pltpu.sync_copy(z_h.at[pl.ds(r0, npt)], acc_s.at[pl.ds(r0, npt)])
        pltpu.sync_copy(att_h, att_v)
        plsc.subcore_barrier()
        iota = lax.iota(jnp.int32, 16)
        zero16 = jnp.zeros((16,), jnp.float32)
        one0 = jnp.where(iota == 0, 1.0, 0.0).astype(jnp.float32)

        def ebase(bb):
            return pl.multiple_of(wid * ept + bb * bsz, 8)

        def issue_idx(s, bb):
            eb = ebase(bb)
            pltpu.async_copy(src_h.at[pl.ds(eb, bsz)], sidx[s], semi[s])
            pltpu.async_copy(dst_h.at[pl.ds(eb, bsz)], didx[s], semi[s])

        def wait_idx(s):
            pltpu.make_async_copy(src_h.at[pl.ds(0, bsz)], sidx[s], semi[s]).wait()
            pltpu.make_async_copy(dst_h.at[pl.ds(0, bsz)], didx[s], semi[s]).wait()

        def issue_gather(s, bb):
            eb = ebase(bb)
            pltpu.async_copy(xl_h.at[sidx[s]], xlb[s], semxl[s])
            pltpu.async_copy(xr_h.at[didx[s]], xrb[s], semxr[s])
            pltpu.async_copy(me_h.at[pl.ds(eb, bsz)], meb[s], semme[s])

        def wait_gather(s):
            pltpu.make_async_copy(xl_h.at[sidx[s]], xlb[s], semxl[s]).wait()
            pltpu.make_async_copy(xr_h.at[didx[s]], xrb[s], semxr[s]).wait()
            pltpu.make_async_copy(me_h.at[pl.ds(0, bsz)], meb[s], semme[s]).wait()

        def issue_scatter(s):
            pltpu.async_copy(con[s], acc_s.at[didx[s]], semsc[s], add=True)

        def wait_scatter(s):
            pltpu.make_async_copy(con[s], acc_s.at[didx[s]], semsc[s]).wait()

        def compute(s):
            xl_v, xr_v, me_v, con_v = xlb[s], xrb[s], meb[s], con[s]
            nch = F // 16
            att_c = [att_v[pl.ds(kk * 16, 16)] for kk in range(nch)]
            @functools.partial(plsc.parallel_loop, 0, bsz, unroll=4)
            def _(r):
                xs = [xl_v[r, pl.ds(kk * 16, 16)] for kk in range(nch)]
                acc = zero16
                for kk in range(nch):
                    mm = (xs[kk] + xr_v[r, pl.ds(kk * 16, 16)]
                          + me_v[r, pl.ds(kk * 16, 16)])
                    mm = jnp.where(mm > 0, mm, 0.2 * mm)
                    acc = acc + mm * att_c[kk]
                alpha = jnp.sum(acc)
                bc = jnp.exp(jnp.broadcast_to(alpha, (16,)))
                for kk in range(nch):
                    con_v[r, pl.ds(kk * 16, 16)] = xs[kk] * bc
                con_v[r, pl.ds(F, 16)] = bc * one0

        pltpu.sync_copy(src_h.at[pl.ds(ebase(0), bsz)], sidx[0])
        pltpu.sync_copy(dst_h.at[pl.ds(ebase(0), bsz)], didx[0])
        issue_gather(0, 0)
        issue_idx(1, 1)
        wait_gather(0)
        compute(0)
        issue_scatter(0)
        wait_idx(1)
        issue_gather(1, 1)

        def pair(i, _):
            bb = 2 * i + 2
            wait_scatter(0)
            issue_idx(0, bb)
            wait_gather(1)
            compute(1)
            issue_scatter(1)
            wait_idx(0)
            issue_gather(0, bb)
            wait_scatter(1)

            @pl.when(bb + 1 < nb)
            def _():
                issue_idx(1, bb + 1)

            wait_gather(0)
            compute(0)
            issue_scatter(0)

            @pl.when(bb + 1 < nb)
            def _():
                wait_idx(1)
                issue_gather(1, bb + 1)

            return 0

        lax.fori_loop(0, (nb - 1) // 2, pair, 0)
        wait_scatter(0)
        plsc.subcore_barrier()
        pltpu.sync_copy(acc_s.at[pl.ds(r0, npt)], out_h.at[cid, pl.ds(r0, npt)])

    return k(xl, xr, me, src, dst, att, zeros)


def _gather_rows(tab, idx):
    kn = idx.shape[0]
    kpt = kn // NW

    @functools.partial(
        pl.kernel,
        out_type=jax.ShapeDtypeStruct((kn, 16), jnp.float32),
        mesh=_mesh(),
        compiler_params=pltpu.CompilerParams(needs_layout_passes=False, use_tc_tiling_on_sc=False),
        scratch_types=[
            pltpu.VMEM((kpt,), jnp.int32),
            pltpu.VMEM((kpt, 16), jnp.float32),
            pltpu.SemaphoreType.DMA,
        ],
    )
    def k(tab_h, idx_h, out_h, idx_v, rows_v, sem):
        cid = lax.axis_index("c")
        sid = lax.axis_index("s")
        base = pl.multiple_of((cid * NS + sid) * kpt, 8)
        pltpu.sync_copy(idx_h.at[pl.ds(base, kpt)], idx_v)
        pltpu.async_copy(tab_h.at[idx_v], rows_v, sem).wait()
        pltpu.sync_copy(rows_v, out_h.at[pl.ds(base, kpt)])

    return k(tab, idx)


def kernel(x, edge_index, edge_attr, y, node_idx,
           Wl1, bl1, Wr1, br1, We1, att1, bias1,
           Wl2, bl2, Wr2, br2, We2, att2, bias2,
           W0, b0, W1, b1, W2, b2):
    n = x.shape[0]
    npad = ((n + 8 * NS - 1) // (8 * NS)) * (8 * NS)
    src = edge_index[0].astype(jnp.int32)
    dst = edge_index[1].astype(jnp.int32)
    node_idx = node_idx.astype(jnp.int32)

    zeros_cw = jnp.zeros((npad, CW), jnp.float32)
    zeros_pw = jnp.zeros((npad, PW), jnp.float32)

    xl1, xr1 = _proj(x, Wl1, bl1, Wr1, br1)
    me1, me2 = _me2(edge_attr, We1, We2)
    pacc = _p0(dst, edge_attr, zeros_pw, npad)
    lme1, lme2 = _loopme(pacc, We1, We2, n)

    acc1 = _edge_pass(xl1, xr1, me1, src, dst, att1, zeros_cw)
    xl2, xr2 = _epi(acc1, xl1, xr1, lme1, att1, bias1, Wl2, bl2, Wr2, br2)
    acc2 = _edge_pass(xl2, xr2, me2, src, dst, att2, zeros_cw)
    outp = _head(acc2, xl2, xr2, lme2, att2, bias2, y, W0, b0, W1, b1, W2, b2)
    sel = _gather_rows(outp, node_idx)
    return sel[:, :8]
